# baseline (speedup 1.0000x reference)
import jax
import jax.numpy as jnp
from jax import lax
from jax.experimental import pallas as pl
from jax.experimental.pallas import tpu as pltpu

V = 100000
D = 64
H = 128
L = 200

NV = 25
R = V // NV


def _body(idx_sref, emb_ref, w1_ref, b1_ref, w2_ref, b2_ref,
          proj_ref, out_ref, h_ref, rows_ref, sem):
    i = pl.program_id(0)

    @pl.when(i == 0)
    def _():
        def start(j, _):
            pltpu.make_async_copy(emb_ref.at[idx_sref[j]], rows_ref.at[j],
                                  sem).start()
            return 0
        lax.fori_loop(0, L, start, 0)

        def drain(j, _):
            pltpu.make_async_copy(emb_ref.at[idx_sref[j]], rows_ref.at[j],
                                  sem).wait()
            return 0
        lax.fori_loop(0, L, drain, 0)

        e = jnp.sum(rows_ref[...], axis=0, keepdims=True)
        pre = lax.dot_general(e, w1_ref[...], (((1,), (1,)), ((), ())),
                              preferred_element_type=jnp.float32)
        h = jnp.maximum(pre + b1_ref[...], 0.0)
        h_ref[...] = h
        proj_ref[...] = h

    h = h_ref[...].astype(jnp.bfloat16)
    w2b = w2_ref[0].astype(jnp.bfloat16)
    lg = lax.dot_general(h, w2b, (((1,), (1,)), ((), ())),
                         preferred_element_type=jnp.float32) + b2_ref[0]
    out_ref[pl.ds(i, 1), :] = lg

    @pl.when(i == NV - 1)
    def _():
        allv = out_ref[...]
        m = jnp.max(allv)
        lse = m + jnp.log(jnp.sum(jnp.exp(allv - m)))
        out_ref[...] = allv - lse


_tc_fused = pl.pallas_call(
    _body,
    grid_spec=pltpu.PrefetchScalarGridSpec(
        num_scalar_prefetch=1,
        grid=(NV,),
        in_specs=[
            pl.BlockSpec(memory_space=pl.ANY),
            pl.BlockSpec((H, D), lambda i, s: (0, 0)),
            pl.BlockSpec((1, H), lambda i, s: (0, 0)),
            pl.BlockSpec((1, R, H), lambda i, s: (i, 0, 0)),
            pl.BlockSpec((1, 1, R), lambda i, s: (i, 0, 0)),
        ],
        out_specs=[
            pl.BlockSpec((1, H), lambda i, s: (0, 0)),
            pl.BlockSpec((NV, R), lambda i, s: (0, 0)),
        ],
        scratch_shapes=[
            pltpu.VMEM((1, H), jnp.float32),
            pltpu.VMEM((L, D), jnp.float32),
            pltpu.SemaphoreType.DMA,
        ],
    ),
    out_shape=[
        jax.ShapeDtypeStruct((1, H), jnp.float32),
        jax.ShapeDtypeStruct((NV, R), jnp.float32),
    ],
)


def kernel(_inputs, emb, W1, b1, W2, b2):
    idx = _inputs.astype(jnp.int32)
    proj, outr = _tc_fused(idx, emb, W1, b1.reshape(1, H),
                           W2.reshape(NV, R, H), b2.reshape(NV, 1, R))
    return (proj, outr.reshape(1, V))


# speedup vs baseline: 1.0070x; 1.0070x over previous
import jax
import jax.numpy as jnp
from jax import lax
from jax.experimental import pallas as pl
from jax.experimental.pallas import tpu as pltpu

V = 100000
D = 64
H = 128
L = 200

NV = 25
R = V // NV
NSEM = 8


def _body(idx_sref, emb_ref, w1_ref, b1_ref, w2_ref, b2_ref,
          proj_ref, out_ref, h_ref, rows_ref, sem):
    i = pl.program_id(0)

    @pl.when(i == 0)
    def _():
        for j in range(L):
            pltpu.make_async_copy(emb_ref.at[idx_sref[j]], rows_ref.at[j],
                                  sem.at[j % NSEM]).start()
        for j in range(L):
            pltpu.make_async_copy(emb_ref.at[idx_sref[j]], rows_ref.at[j],
                                  sem.at[j % NSEM]).wait()

        e = jnp.sum(rows_ref[...], axis=0, keepdims=True)
        pre = lax.dot_general(e, w1_ref[...], (((1,), (1,)), ((), ())),
                              preferred_element_type=jnp.float32)
        h = jnp.maximum(pre + b1_ref[...], 0.0)
        h_ref[...] = h
        proj_ref[...] = h

    h = h_ref[...].astype(jnp.bfloat16)
    w2b = w2_ref[0].astype(jnp.bfloat16)
    lg = lax.dot_general(h, w2b, (((1,), (1,)), ((), ())),
                         preferred_element_type=jnp.float32) + b2_ref[0]
    out_ref[pl.ds(i, 1), :] = lg

    @pl.when(i == NV - 1)
    def _():
        allv = out_ref[...]
        m = jnp.max(allv)
        lse = m + jnp.log(jnp.sum(jnp.exp(allv - m)))
        out_ref[...] = allv - lse


_tc_fused = pl.pallas_call(
    _body,
    grid_spec=pltpu.PrefetchScalarGridSpec(
        num_scalar_prefetch=1,
        grid=(NV,),
        in_specs=[
            pl.BlockSpec(memory_space=pl.ANY),
            pl.BlockSpec((H, D), lambda i, s: (0, 0)),
            pl.BlockSpec((1, H), lambda i, s: (0, 0)),
            pl.BlockSpec((1, R, H), lambda i, s: (i, 0, 0)),
            pl.BlockSpec((1, 1, R), lambda i, s: (i, 0, 0)),
        ],
        out_specs=[
            pl.BlockSpec((1, H), lambda i, s: (0, 0)),
            pl.BlockSpec((NV, R), lambda i, s: (0, 0)),
        ],
        scratch_shapes=[
            pltpu.VMEM((1, H), jnp.float32),
            pltpu.VMEM((L, D), jnp.float32),
            pltpu.SemaphoreType.DMA((NSEM,)),
        ],
    ),
    out_shape=[
        jax.ShapeDtypeStruct((1, H), jnp.float32),
        jax.ShapeDtypeStruct((NV, R), jnp.float32),
    ],
)


def kernel(_inputs, emb, W1, b1, W2, b2):
    idx = _inputs.astype(jnp.int32)
    proj, outr = _tc_fused(idx, emb, W1, b1.reshape(1, H),
                           W2.reshape(NV, R, H), b2.reshape(NV, 1, R))
    return (proj, outr.reshape(1, V))
